# baseline (speedup 1.0000x reference)
import functools

import jax
import jax.numpy as jnp
from jax import lax
from jax.experimental import pallas as pl
from jax.experimental.pallas import tpu as pltpu
from jax.experimental.pallas import tpu_sc as plsc

L = 16
NC = 2
NS = 16
KB = 512
KU = 1
ZR = 6256
RB = 800


def _sc_mesh():
    return plsc.VectorSubcoreMesh(
        core_axis_name="c", subcore_axis_name="s", num_cores=NC, num_subcores=NS
    )


def _wb_rows(n):
    return ((n + NS * 8 - 1) // (NS * 8)) * 8


def _make_deg(n, rows2d):
    rpt = rows2d // (NC * NS)
    nch = rpt // KU
    wb = _wb_rows(n)
    n_pad = NS * wb
    racc = NS * ZR

    @functools.partial(
        pl.kernel,
        out_type=jax.ShapeDtypeStruct((2 * n_pad, L), jnp.float32),
        mesh=_sc_mesh(),
        scratch_types=[
            pltpu.VMEM((KU, KB), jnp.int32),
            pltpu.VMEM((KB, L), jnp.float32),
            pltpu.VMEM_SHARED((racc, L), jnp.float32),
        ],
        compiler_params=pltpu.CompilerParams(use_tc_tiling_on_sc=False),
    )
    def deg_kernel(dst2, zrs, ones_h, out, didx, ones_v, acc):
        c = lax.axis_index("c")
        s = lax.axis_index("s")
        pltpu.sync_copy(ones_h, ones_v)
        pltpu.sync_copy(zrs, acc.at[pl.ds(s * ZR, ZR)])
        plsc.subcore_barrier()

        def body(i, carry):
            r0 = c * (rpt * NS) + s * rpt + i * KU
            pltpu.sync_copy(dst2.at[pl.ds(r0, KU)], didx)
            for j in range(KU):
                pltpu.sync_copy(ones_v, acc.at[didx.at[j]], add=True)
            return carry

        lax.fori_loop(0, nch, body, None)
        plsc.subcore_barrier()
        pltpu.sync_copy(acc.at[pl.ds(s * wb, wb)],
                        out.at[pl.ds(c * n_pad + s * wb, wb)])

    return deg_kernel


def _make_conv(g_total, n, rows2d):
    rpt = rows2d // NS
    nch = rpt // KU
    gh = g_total // NC
    wb = _wb_rows(n)
    n_pad = NS * wb
    racc = NS * ZR

    @functools.partial(
        pl.kernel,
        out_type=jax.ShapeDtypeStruct((g_total * n_pad, L), jnp.float32),
        mesh=_sc_mesh(),
        scratch_types=[
            pltpu.VMEM((KU, KB), jnp.int32),
            pltpu.VMEM((KU, KB), jnp.int32),
            pltpu.VMEM((KU, KB), jnp.int32),
            pltpu.VMEM((KU, KB), jnp.int32),
            pltpu.VMEM((KU, KB, L), jnp.float32),
            pltpu.VMEM((KU, KB, L), jnp.float32),
            pltpu.SemaphoreType.DMA,
            pltpu.SemaphoreType.DMA,
            pltpu.SemaphoreType.DMA,
            pltpu.SemaphoreType.DMA,
            pltpu.VMEM_SHARED((racc, L), jnp.float32),
        ],
        compiler_params=pltpu.CompilerParams(use_tc_tiling_on_sc=False),
    )
    def conv_kernel(tbl, src2, dst2, out,
                    sidx0, didx0, sidx1, didx1, rows0, rows1,
                    gsem0, gsem1, isem0, isem1, acc):
        c = lax.axis_index("c")
        s = lax.axis_index("s")
        base = s * rpt

        def fire_idx(i_clamped, sref, dref, sem):
            r0 = base + i_clamped * KU
            pltpu.async_copy(src2.at[pl.ds(r0, KU)], sref, sem)
            pltpu.async_copy(dst2.at[pl.ds(r0, KU)], dref, sem)

        def wait_idx(sref, dref, sem):
            pltpu.make_async_copy(src2.at[pl.ds(base, KU)], sref, sem).wait()
            pltpu.make_async_copy(dst2.at[pl.ds(base, KU)], dref, sem).wait()

        def fire_gathers(tblg, sref, rbuf, sem):
            for j in range(KU):
                pltpu.async_copy(tblg.at[sref.at[j]], rbuf.at[j], sem)

        def wait_gathers(tblg, sref, rbuf, sem):
            for j in range(KU):
                pltpu.make_async_copy(tblg.at[sref.at[j]], rbuf.at[j],
                                      sem).wait()

        def scatter(dref, rbuf):
            for j in range(KU):
                pltpu.sync_copy(rbuf.at[j], acc.at[dref.at[j]], add=True)

        for p in range(gh):
            g = c * gh + p
            tblg = tbl.at[g]
            pltpu.sync_copy(tblg.at[pl.ds(s * wb, wb)], acc.at[pl.ds(s * wb, wb)])
            plsc.subcore_barrier()

            pltpu.sync_copy(src2.at[pl.ds(base, KU)], sidx0)
            pltpu.sync_copy(dst2.at[pl.ds(base, KU)], didx0)
            fire_gathers(tblg, sidx0, rows0, gsem0)
            fire_idx(jnp.int32(1), sidx1, didx1, isem1)

            def body(k, carry):
                i2 = jnp.minimum(2 * k + 2, nch - 1)
                i3 = jnp.minimum(2 * k + 3, nch - 1)
                wait_idx(sidx1, didx1, isem1)
                fire_gathers(tblg, sidx1, rows1, gsem1)
                wait_gathers(tblg, sidx0, rows0, gsem0)
                scatter(didx0, rows0)
                fire_idx(i2, sidx0, didx0, isem0)
                wait_idx(sidx0, didx0, isem0)
                fire_gathers(tblg, sidx0, rows0, gsem0)
                wait_gathers(tblg, sidx1, rows1, gsem1)
                scatter(didx1, rows1)
                fire_idx(i3, sidx1, didx1, isem1)
                return carry

            lax.fori_loop(0, nch // 2, body, None)
            wait_gathers(tblg, sidx0, rows0, gsem0)
            wait_idx(sidx1, didx1, isem1)

            plsc.subcore_barrier()
            ob = g * n_pad + s * wb
            pltpu.sync_copy(acc.at[pl.ds(s * wb, wb)], out.at[pl.ds(ob, wb)])
            plsc.subcore_barrier()

    return conv_kernel


def _tc_prep(x, w1, degn, n):
    f_in = x.shape[1]
    d1 = w1.shape[1]
    grid = n // RB

    def body(x_ref, w_ref, d0_ref, d1_ref, hn_ref, dinv_ref):
        deg = d0_ref[0] + d1_ref[0] + 1.0
        dinv = lax.rsqrt(deg)
        h = jnp.dot(x_ref[...], w_ref[...], preferred_element_type=jnp.float32)
        hn_ref[...] = h * dinv
        dinv_ref[...] = dinv

    return pl.pallas_call(
        body,
        grid=(grid,),
        in_specs=[
            pl.BlockSpec((RB, f_in), lambda i: (i, 0)),
            pl.BlockSpec((f_in, d1), lambda i: (0, 0)),
            pl.BlockSpec((1, RB, 1), lambda i: (0, i, 0)),
            pl.BlockSpec((1, RB, 1), lambda i: (1, i, 0)),
        ],
        out_specs=[
            pl.BlockSpec((RB, d1), lambda i: (i, 0)),
            pl.BlockSpec((RB, 1), lambda i: (i, 0)),
        ],
        out_shape=[
            jax.ShapeDtypeStruct((n, d1), jnp.float32),
            jax.ShapeDtypeStruct((n, 1), jnp.float32),
        ],
    )(x, w1, degn, degn)


def _tc_mid(acc, dinv, b1r, w2, n):
    d1 = acc.shape[1]
    d2 = w2.shape[1]
    grid = n // RB

    def body(a_ref, dinv_ref, b_ref, w_ref, out_ref):
        z = jnp.maximum(a_ref[...] * dinv_ref[...] + b_ref[...], 0.0)
        h2 = jnp.dot(z, w_ref[...], preferred_element_type=jnp.float32)
        out_ref[...] = h2 * dinv_ref[...]

    return pl.pallas_call(
        body,
        grid=(grid,),
        in_specs=[
            pl.BlockSpec((RB, d1), lambda i: (i, 0)),
            pl.BlockSpec((RB, 1), lambda i: (i, 0)),
            pl.BlockSpec((1, d1), lambda i: (0, 0)),
            pl.BlockSpec((d1, d2), lambda i: (0, 0)),
        ],
        out_specs=pl.BlockSpec((RB, d2), lambda i: (i, 0)),
        out_shape=jax.ShapeDtypeStruct((n, d2), jnp.float32),
    )(acc, dinv, b1r, w2)


def _tc_final(acc, dinv, b2r, n):
    d2 = acc.shape[1]
    grid = n // RB

    def body(a_ref, dinv_ref, b_ref, out_ref):
        o = a_ref[...] * dinv_ref[...] + b_ref[...]
        m = jnp.max(o, axis=1, keepdims=True)
        lse = jnp.log(jnp.sum(jnp.exp(o - m), axis=1, keepdims=True)) + m
        out_ref[...] = o - lse

    return pl.pallas_call(
        body,
        grid=(grid,),
        in_specs=[
            pl.BlockSpec((RB, d2), lambda i: (i, 0)),
            pl.BlockSpec((RB, 1), lambda i: (i, 0)),
            pl.BlockSpec((1, d2), lambda i: (0, 0)),
        ],
        out_specs=pl.BlockSpec((RB, d2), lambda i: (i, 0)),
        out_shape=jax.ShapeDtypeStruct((n, d2), jnp.float32),
    )(acc, dinv, b2r)


def _to_groups(h_nm, n, n_pad):
    g = h_nm.shape[1] // L
    hp = jnp.pad(h_nm, ((0, n_pad - n), (0, 0)))
    return hp.reshape(n_pad, g, L).transpose(1, 0, 2)


def _from_groups(acc_flat, n, n_pad, d):
    g = d // L
    return (acc_flat.reshape(g, n_pad, L)[:, :n]
            .transpose(1, 0, 2).reshape(n, d))


def kernel(x, edge_index, W1, b1, W2, b2):
    n = x.shape[0]
    e = edge_index.shape[1]
    d1 = W1.shape[1]
    d2 = W2.shape[1]
    g1, g2 = d1 // L, d2 // L

    ei = edge_index.astype(jnp.int32)
    src, dst = ei[0], ei[1]

    grain = 2 * NS * 2 * KU * KB
    e_pad = ((e + grain - 1) // grain) * grain
    pad = e_pad - e
    srcp = jnp.concatenate([src, jnp.zeros((pad,), jnp.int32)])
    dstp = jnp.concatenate([dst, jnp.full((pad,), n, jnp.int32)])
    src2 = srcp.reshape(-1, KB)
    dst2 = dstp.reshape(-1, KB)
    rows2d = src2.shape[0]

    zrs = jnp.zeros((ZR, L), jnp.float32)
    ones_h = jnp.ones((KB, L), jnp.float32)

    n_pad = NS * _wb_rows(n)

    degf = _make_deg(n, rows2d)(dst2, zrs, ones_h)
    degn = degf.reshape(2, n_pad, L)[:, :n, :1]

    hn1, dinv = _tc_prep(x, W1, degn, n)

    acc1 = _make_conv(g1, n, rows2d)(_to_groups(hn1, n, n_pad), src2, dst2)
    hn2 = _tc_mid(_from_groups(acc1, n, n_pad, d1), dinv,
                  b1.reshape(1, d1), W2, n)

    acc2 = _make_conv(g2, n, rows2d)(_to_groups(hn2, n, n_pad), src2, dst2)
    return _tc_final(_from_groups(acc2, n, n_pad, d2), dinv,
                     b2.reshape(1, d2), n)


# speedup vs baseline: 1.1872x; 1.1872x over previous
import functools

import jax
import jax.numpy as jnp
from jax import lax
from jax.experimental import pallas as pl
from jax.experimental.pallas import tpu as pltpu
from jax.experimental.pallas import tpu_sc as plsc

L = 16
NC = 2
NS = 16
KB = 512
KU = 1
ZR = 6256
RB = 800


def _sc_mesh():
    return plsc.VectorSubcoreMesh(
        core_axis_name="c", subcore_axis_name="s", num_cores=NC, num_subcores=NS
    )


def _wb_rows(n):
    return ((n + NS * 8 - 1) // (NS * 8)) * 8


def _make_deg(n, rows2d):
    rpt = rows2d // (NC * NS)
    nch = rpt // KU
    wb = _wb_rows(n)
    n_pad = NS * wb
    racc = NS * ZR

    @functools.partial(
        pl.kernel,
        out_type=jax.ShapeDtypeStruct((2 * n_pad, L), jnp.float32),
        mesh=_sc_mesh(),
        scratch_types=[
            pltpu.VMEM((KU, KB), jnp.int32),
            pltpu.VMEM((KB, L), jnp.float32),
            pltpu.VMEM_SHARED((racc, L), jnp.float32),
        ],
        compiler_params=pltpu.CompilerParams(use_tc_tiling_on_sc=False),
    )
    def deg_kernel(dst2, zrs, ones_h, out, didx, ones_v, acc):
        c = lax.axis_index("c")
        s = lax.axis_index("s")
        pltpu.sync_copy(ones_h, ones_v)
        pltpu.sync_copy(zrs, acc.at[pl.ds(s * ZR, ZR)])
        plsc.subcore_barrier()

        def body(i, carry):
            r0 = c * (rpt * NS) + s * rpt + i * KU
            pltpu.sync_copy(dst2.at[pl.ds(r0, KU)], didx)
            for j in range(KU):
                pltpu.sync_copy(ones_v, acc.at[didx.at[j]], add=True)
            return carry

        lax.fori_loop(0, nch, body, None)
        plsc.subcore_barrier()
        pltpu.sync_copy(acc.at[pl.ds(s * wb, wb)],
                        out.at[pl.ds(c * n_pad + s * wb, wb)])

    return deg_kernel


def _make_conv(g_total, n, rows2d):
    rpt = rows2d // NS
    nch = rpt // KU
    gh = g_total // NC
    wb = _wb_rows(n)
    n_pad = NS * wb
    racc = NS * ZR

    @functools.partial(
        pl.kernel,
        out_type=jax.ShapeDtypeStruct((g_total * n_pad, L), jnp.float32),
        mesh=_sc_mesh(),
        scratch_types=[
            pltpu.VMEM((KU, KB), jnp.int32),
            pltpu.VMEM((KU, KB), jnp.int32),
            pltpu.VMEM((KU, KB), jnp.int32),
            pltpu.VMEM((KU, KB), jnp.int32),
            pltpu.VMEM((KU, KB, L), jnp.float32),
            pltpu.VMEM((KU, KB, L), jnp.float32),
            pltpu.SemaphoreType.DMA,
            pltpu.SemaphoreType.DMA,
            pltpu.SemaphoreType.DMA,
            pltpu.SemaphoreType.DMA,
            pltpu.VMEM_SHARED((racc, L), jnp.float32),
        ],
        compiler_params=pltpu.CompilerParams(use_tc_tiling_on_sc=False),
    )
    def conv_kernel(tbl, src2, dst2, out,
                    sidx0, didx0, sidx1, didx1, rows0, rows1,
                    gsem0, gsem1, isem0, isem1, acc):
        c = lax.axis_index("c")
        s = lax.axis_index("s")
        base = s * rpt

        def fire_idx(i_clamped, sref, dref, sem):
            r0 = base + i_clamped * KU
            pltpu.async_copy(src2.at[pl.ds(r0, KU)], sref, sem)
            pltpu.async_copy(dst2.at[pl.ds(r0, KU)], dref, sem)

        def wait_idx(sref, dref, sem):
            pltpu.make_async_copy(src2.at[pl.ds(base, KU)], sref, sem).wait()
            pltpu.make_async_copy(dst2.at[pl.ds(base, KU)], dref, sem).wait()

        def fire_gathers(tblg, sref, rbuf, sem):
            for j in range(KU):
                pltpu.async_copy(tblg.at[sref.at[j]], rbuf.at[j], sem)

        def wait_gathers(tblg, sref, rbuf, sem):
            for j in range(KU):
                pltpu.make_async_copy(tblg.at[sref.at[j]], rbuf.at[j],
                                      sem).wait()

        def scatter(dref, rbuf):
            for j in range(KU):
                pltpu.sync_copy(rbuf.at[j], acc.at[dref.at[j]], add=True)

        for p in range(gh):
            g = c * gh + p
            tblg = tbl.at[g]
            pltpu.sync_copy(tblg.at[pl.ds(s * wb, wb)], acc.at[pl.ds(s * wb, wb)])
            plsc.subcore_barrier()

            pltpu.sync_copy(src2.at[pl.ds(base, KU)], sidx0)
            pltpu.sync_copy(dst2.at[pl.ds(base, KU)], didx0)
            fire_gathers(tblg, sidx0, rows0, gsem0)
            fire_idx(jnp.int32(1), sidx1, didx1, isem1)

            def body(k, carry):
                i2 = jnp.minimum(2 * k + 2, nch - 1)
                i3 = jnp.minimum(2 * k + 3, nch - 1)
                wait_idx(sidx1, didx1, isem1)
                fire_gathers(tblg, sidx1, rows1, gsem1)
                wait_gathers(tblg, sidx0, rows0, gsem0)
                scatter(didx0, rows0)
                fire_idx(i2, sidx0, didx0, isem0)
                wait_idx(sidx0, didx0, isem0)
                fire_gathers(tblg, sidx0, rows0, gsem0)
                wait_gathers(tblg, sidx1, rows1, gsem1)
                scatter(didx1, rows1)
                fire_idx(i3, sidx1, didx1, isem1)
                return carry

            lax.fori_loop(0, nch // 2, body, None)
            wait_gathers(tblg, sidx0, rows0, gsem0)
            wait_idx(sidx1, didx1, isem1)

            plsc.subcore_barrier()
            ob = g * n_pad + s * wb
            pltpu.sync_copy(acc.at[pl.ds(s * wb, wb)], out.at[pl.ds(ob, wb)])
            plsc.subcore_barrier()

    return conv_kernel


def _tc_prep(x, w1, degr, n, n_pad):
    f_in = x.shape[1]
    d1 = w1.shape[1]
    g1 = d1 // L
    grid = n // RB

    def body(x_ref, w_ref, d0_ref, d1_ref, hng_ref, dinv_ref):
        deg = d0_ref[0, :, 0:1] + d1_ref[0, :, 0:1] + 1.0
        dinv = lax.rsqrt(deg)
        h = jnp.dot(x_ref[...], w_ref[...], preferred_element_type=jnp.float32)
        hn = h * dinv
        hng_ref[...] = jnp.stack([hn[:, g * L:(g + 1) * L] for g in range(g1)], 0)
        dinv_ref[...] = dinv

    return pl.pallas_call(
        body,
        grid=(grid,),
        in_specs=[
            pl.BlockSpec((RB, f_in), lambda i: (i, 0)),
            pl.BlockSpec((f_in, d1), lambda i: (0, 0)),
            pl.BlockSpec((1, RB, L), lambda i: (0, i, 0)),
            pl.BlockSpec((1, RB, L), lambda i: (1, i, 0)),
        ],
        out_specs=[
            pl.BlockSpec((g1, RB, L), lambda i: (0, i, 0)),
            pl.BlockSpec((RB, 1), lambda i: (i, 0)),
        ],
        out_shape=[
            jax.ShapeDtypeStruct((g1, n_pad, L), jnp.float32),
            jax.ShapeDtypeStruct((n, 1), jnp.float32),
        ],
    )(x, w1, degr, degr)


def _tc_mid(accr, dinv, b1r, w2, n, n_pad):
    g1 = accr.shape[0]
    d1 = g1 * L
    d2 = w2.shape[1]
    g2 = d2 // L
    grid = n // RB

    def body(a_ref, dinv_ref, b_ref, w_ref, out_ref):
        tot = a_ref[...]
        cat = jnp.concatenate([tot[g] for g in range(g1)], axis=1)
        z = jnp.maximum(cat * dinv_ref[...] + b_ref[...], 0.0)
        h2 = jnp.dot(z, w_ref[...], preferred_element_type=jnp.float32)
        hn2 = h2 * dinv_ref[...]
        out_ref[...] = jnp.stack([hn2[:, g * L:(g + 1) * L] for g in range(g2)], 0)

    return pl.pallas_call(
        body,
        grid=(grid,),
        in_specs=[
            pl.BlockSpec((g1, RB, L), lambda i: (0, i, 0)),
            pl.BlockSpec((RB, 1), lambda i: (i, 0)),
            pl.BlockSpec((1, d1), lambda i: (0, 0)),
            pl.BlockSpec((d1, d2), lambda i: (0, 0)),
        ],
        out_specs=pl.BlockSpec((g2, RB, L), lambda i: (0, i, 0)),
        out_shape=jax.ShapeDtypeStruct((g2, n_pad, L), jnp.float32),
    )(accr, dinv, b1r, w2)


def _tc_final(accr, dinv, b2r, n):
    g2 = accr.shape[0]
    d2 = g2 * L
    grid = n // RB

    def body(a_ref, dinv_ref, b_ref, out_ref):
        tot = a_ref[...]
        o = jnp.concatenate([tot[g] for g in range(g2)], axis=1)
        o = o * dinv_ref[...] + b_ref[...]
        m = jnp.max(o, axis=1, keepdims=True)
        lse = jnp.log(jnp.sum(jnp.exp(o - m), axis=1, keepdims=True)) + m
        out_ref[...] = o - lse

    return pl.pallas_call(
        body,
        grid=(grid,),
        in_specs=[
            pl.BlockSpec((g2, RB, L), lambda i: (0, i, 0)),
            pl.BlockSpec((RB, 1), lambda i: (i, 0)),
            pl.BlockSpec((1, d2), lambda i: (0, 0)),
        ],
        out_specs=pl.BlockSpec((RB, d2), lambda i: (i, 0)),
        out_shape=jax.ShapeDtypeStruct((n, d2), jnp.float32),
    )(accr, dinv, b2r)


def kernel(x, edge_index, W1, b1, W2, b2):
    n = x.shape[0]
    e = edge_index.shape[1]
    d1 = W1.shape[1]
    d2 = W2.shape[1]
    g1, g2 = d1 // L, d2 // L

    ei = edge_index.astype(jnp.int32)
    src, dst = ei[0], ei[1]

    grain = 2 * NS * 2 * KU * KB
    e_pad = ((e + grain - 1) // grain) * grain
    pad = e_pad - e
    srcp = jnp.concatenate([src, jnp.zeros((pad,), jnp.int32)])
    dstp = jnp.concatenate([dst, jnp.full((pad,), n, jnp.int32)])
    src2 = srcp.reshape(-1, KB)
    dst2 = dstp.reshape(-1, KB)
    rows2d = src2.shape[0]

    zrs = jnp.zeros((ZR, L), jnp.float32)
    ones_h = jnp.ones((KB, L), jnp.float32)

    n_pad = NS * _wb_rows(n)

    degf = _make_deg(n, rows2d)(dst2, zrs, ones_h)
    degr = degf.reshape(2, n_pad, L)

    hng1, dinv = _tc_prep(x, W1, degr, n, n_pad)

    acc1 = _make_conv(g1, n, rows2d)(hng1, src2, dst2)
    hng2 = _tc_mid(acc1.reshape(g1, n_pad, L), dinv,
                   b1.reshape(1, d1), W2, n, n_pad)

    acc2 = _make_conv(g2, n, rows2d)(hng2, src2, dst2)
    return _tc_final(acc2.reshape(g2, n_pad, L), dinv,
                     b2.reshape(1, d2), n)
